# baseline (speedup 1.0000x reference)
import functools

import jax
import jax.numpy as jnp
from jax import lax
from jax.experimental import pallas as pl
from jax.experimental.pallas import tpu as pltpu
from jax.experimental.pallas import tpu_sc as plsc

_NC = 2
_NS = 16
_NW = _NC * _NS
_BL = 4096
_SLOTS = _BL // _NW
_R = 8
_NDMA = _SLOTS // _R


def _sc_body(emb_hbm, out_hbm, stage_v, sem):
    cid = lax.axis_index("c")
    sid = lax.axis_index("s")
    wid = sid * _NC + cid
    base = wid * _SLOTS
    for r in range(_R):
        pltpu.sync_copy(emb_hbm, stage_v.at[r])
    for j in range(_NDMA):
        pltpu.make_async_copy(
            stage_v, out_hbm.at[pl.ds(base + j * _R, _R)], sem
        ).start()
    for j in range(_NDMA):
        pltpu.make_async_copy(
            stage_v, out_hbm.at[pl.ds(base + j * _R, _R)], sem
        ).wait()


def kernel(x, var_emb):
    B, L, D = x.shape
    d_model = var_emb.shape[1]
    emb = var_emb[:D]

    sc_call = functools.partial(
        pl.kernel,
        out_type=jax.ShapeDtypeStruct((_BL, D, d_model), var_emb.dtype),
        mesh=plsc.VectorSubcoreMesh(core_axis_name="c", subcore_axis_name="s"),
        scratch_types=[
            pltpu.VMEM((_R, D, d_model), var_emb.dtype),
            pltpu.SemaphoreType.DMA,
        ],
    )(_sc_body)
    out = sc_call(emb)
    return out.reshape(B, L, D, d_model)


# speedup vs baseline: 1.6993x; 1.6993x over previous
import jax
import jax.numpy as jnp
from jax.experimental import pallas as pl

_BLOCK_BL = 256


def _bcast_kernel(emb_ref, out_ref):
    out_ref[...] = jnp.broadcast_to(emb_ref[...][None], out_ref.shape)


def kernel(x, var_emb):
    B, L, D = x.shape
    d_model = var_emb.shape[1]
    BL = B * L
    emb = var_emb[:D]

    out = pl.pallas_call(
        _bcast_kernel,
        grid=(BL // _BLOCK_BL,),
        in_specs=[pl.BlockSpec((D, d_model), lambda i: (0, 0))],
        out_specs=pl.BlockSpec((_BLOCK_BL, D, d_model), lambda i: (i, 0, 0)),
        out_shape=jax.ShapeDtypeStruct((BL, D, d_model), var_emb.dtype),
    )(emb)
    return out.reshape(B, L, D, d_model)
